# baseline (speedup 1.0000x reference)
import functools

import jax
import jax.numpy as jnp
from jax import lax
from jax.experimental import pallas as pl
from jax.experimental.pallas import tpu as pltpu

BB = 320000
FF = 144
DD = 128

ROWS_A = 2560
ROWS_B = 2560


def _a_body(hk_ref, mk_ref, wr1_ref, wr2_ref, br_ref, rm_ref):
    hk = hk_ref[...]
    mk = mk_ref[...]
    acc = (jnp.dot(hk, wr1_ref[...], preferred_element_type=jnp.float32)
           + jnp.dot(mk, wr2_ref[...], preferred_element_type=jnp.float32)
           + br_ref[...])
    rm_ref[...] = jax.nn.sigmoid(acc) * mk


def _dense_rm(h_ki, mess_ki, Wr_w, Wr_b):
    wr1 = Wr_w[:, :FF].T
    wr2 = Wr_w[:, FF:].T
    br = Wr_b.reshape(1, DD)
    nblk = BB // ROWS_A
    return pl.pallas_call(
        _a_body,
        grid=(nblk,),
        in_specs=[
            pl.BlockSpec((ROWS_A, FF), lambda i: (i, 0)),
            pl.BlockSpec((ROWS_A, DD), lambda i: (i, 0)),
            pl.BlockSpec((FF, DD), lambda i: (0, 0)),
            pl.BlockSpec((DD, DD), lambda i: (0, 0)),
            pl.BlockSpec((1, DD), lambda i: (0, 0)),
        ],
        out_specs=pl.BlockSpec((ROWS_A, DD), lambda i: (i, 0)),
        out_shape=jax.ShapeDtypeStruct((BB, DD), jnp.float32),
    )(h_ki, mess_ki, wr1, wr2, br)


def _b_body(h_ref, s_ref, r_ref, wz1_ref, wz2_ref, bz_ref, ww_ref, bw_ref,
            uw_ref, out_ref):
    h = h_ref[...]
    s = s_ref[...]
    r = r_ref[...]
    z = jax.nn.sigmoid(jnp.dot(h, wz1_ref[...], preferred_element_type=jnp.float32)
                       + jnp.dot(s, wz2_ref[...], preferred_element_type=jnp.float32)
                       + bz_ref[...])
    m = jnp.tanh(jnp.dot(h, ww_ref[...], preferred_element_type=jnp.float32)
                 + bw_ref[...]
                 + jnp.dot(r, uw_ref[...], preferred_element_type=jnp.float32))
    out_ref[...] = (1.0 - z) * s + z * m


def _dense_out(h_ij, s_ij, r_ij, Wz_w, Wz_b, U_w, W_w, W_b):
    wz1 = Wz_w[:, :FF].T
    wz2 = Wz_w[:, FF:].T
    bz = Wz_b.reshape(1, DD)
    ww = W_w.T
    bw = W_b.reshape(1, DD)
    uw = U_w.T
    nblk = BB // ROWS_B
    return pl.pallas_call(
        _b_body,
        grid=(nblk,),
        in_specs=[
            pl.BlockSpec((ROWS_B, FF), lambda i: (i, 0)),
            pl.BlockSpec((ROWS_B, DD), lambda i: (i, 0)),
            pl.BlockSpec((ROWS_B, DD), lambda i: (i, 0)),
            pl.BlockSpec((FF, DD), lambda i: (0, 0)),
            pl.BlockSpec((DD, DD), lambda i: (0, 0)),
            pl.BlockSpec((1, DD), lambda i: (0, 0)),
            pl.BlockSpec((FF, DD), lambda i: (0, 0)),
            pl.BlockSpec((1, DD), lambda i: (0, 0)),
            pl.BlockSpec((DD, DD), lambda i: (0, 0)),
        ],
        out_specs=pl.BlockSpec((ROWS_B, DD), lambda i: (i, 0)),
        out_shape=jax.ShapeDtypeStruct((BB, DD), jnp.float32),
    )(h_ij, s_ij, r_ij, wz1, wz2, bz, ww, bw, uw)


def kernel(h_ij, h_ki, mess, src_idx, nei_idx, Wz_w, Wz_b, Wr_w, Wr_b, U_w,
           W_w, W_b):
    mess_ki = jnp.take(mess, nei_idx, axis=0)
    s_ij = jax.ops.segment_sum(mess_ki, src_idx, num_segments=BB)
    rm = _dense_rm(h_ki, mess_ki, Wr_w, Wr_b)
    r_ij = jax.ops.segment_sum(rm, src_idx, num_segments=BB)
    return _dense_out(h_ij, s_ij, r_ij, Wz_w, Wz_b, U_w, W_w, W_b)
